# baseline (speedup 1.0000x reference)
import functools

import jax
import jax.numpy as jnp
from jax import lax
from jax.experimental import pallas as pl
from jax.experimental.pallas import tpu as pltpu
from jax.experimental.pallas import tpu_sc as plsc

N = 100000
E = 1600000

NP = 100352
NA = NP + 128
NT = 16
NW = 32
TZ = NA // NT
TW = NP // NT

EP = 1605632
ER = EP // 128
PT = EP // NW
PR = PT // 128
NB = PR // 8

BE = 8192
BN = 1024
GN = NP // BN

_f32 = jnp.float32


def _edge_mlp_body(ea_ref, w1_ref, b1_ref, w2_ref, b2_ref, ef0_ref, ef1_ref):
    t = jnp.maximum(
        jnp.dot(ea_ref[...], w1_ref[...], preferred_element_type=_f32)
        + b1_ref[...], 0.0)
    ef = jnp.maximum(
        jnp.dot(t, w2_ref[...], preferred_element_type=_f32)
        + b2_ref[...], 0.0)
    ef0_ref[...] = ef[:, :16]
    ef1_ref[...] = ef[:, 16:]


def _edge_mlp(edge_attr, W_e1, b_e1, W_e2p, b_e2p):
    return pl.pallas_call(
        _edge_mlp_body,
        grid=(EP // BE,),
        in_specs=[
            pl.BlockSpec((BE, 16), lambda i: (i, 0)),
            pl.BlockSpec((16, 128), lambda i: (0, 0)),
            pl.BlockSpec((1, 128), lambda i: (0, 0)),
            pl.BlockSpec((128, 32), lambda i: (0, 0)),
            pl.BlockSpec((1, 32), lambda i: (0, 0)),
        ],
        out_specs=[
            pl.BlockSpec((BE, 16), lambda i: (i, 0)),
            pl.BlockSpec((BE, 16), lambda i: (i, 0)),
        ],
        out_shape=[
            jax.ShapeDtypeStruct((EP, 16), _f32),
            jax.ShapeDtypeStruct((EP, 16), _f32),
        ],
    )(edge_attr, W_e1, b_e1, W_e2p, b_e2p)


def _edge_agg_body(ch, src_hbm, col_hbm, ef0_hbm, ef1_hbm, z_hbm, ones_hbm,
                   sp0_hbm, sp1_hbm, degp_hbm,
                   acc, isrc, icol, upd):
    s = lax.axis_index("s")
    w = s * 2 + ch
    base_r = w * PR

    for idx_hbm, ef_hbm, sp_hbm in ((src_hbm, ef0_hbm, sp0_hbm),
                                    (src_hbm, ef1_hbm, sp1_hbm),
                                    (col_hbm, None, degp_hbm)):
        pltpu.sync_copy(z_hbm, acc.at[pl.ds(s * TZ, TZ), :])
        if ef_hbm is None:
            pltpu.sync_copy(ones_hbm, upd)
        plsc.subcore_barrier()

        def batch(g, carry, idx_hbm=idx_hbm, ef_hbm=ef_hbm):
            r0 = base_r + g * 8
            pltpu.sync_copy(idx_hbm.at[pl.ds(r0, 8), :], isrc)
            if ef_hbm is not None:
                pltpu.sync_copy(ef_hbm.at[pl.ds(r0 * 128, 1024), :], upd)
            for j in range(8):
                pltpu.sync_copy(upd.at[pl.ds(j * 128, 128), :],
                                acc.at[isrc.at[j]], add=True)
            return carry

        lax.fori_loop(0, NB, batch, 0)
        plsc.subcore_barrier()
        pltpu.sync_copy(acc.at[pl.ds(s * TW, TW), :],
                        sp_hbm.at[pl.ds(s * TW, TW), :])
        plsc.subcore_barrier()


def _edge_agg(ch, src2, col2, ef0, ef1, zrows, ones1k):
    mesh = plsc.VectorSubcoreMesh(core_axis_name="c", subcore_axis_name="s",
                                  num_cores=1)
    return pl.kernel(
        functools.partial(_edge_agg_body, ch),
        out_type=[
            jax.ShapeDtypeStruct((NP, 16), _f32),
            jax.ShapeDtypeStruct((NP, 16), _f32),
            jax.ShapeDtypeStruct((NP, 16), _f32),
        ],
        mesh=mesh,
        compiler_params=pltpu.CompilerParams(use_tc_tiling_on_sc=False),
        scratch_types=[
            pltpu.VMEM_SHARED((NA, 16), _f32),
            pltpu.VMEM((8, 128), jnp.int32),
            pltpu.VMEM((8, 128), jnp.int32),
            pltpu.VMEM((1024, 16), _f32),
        ],
    )(src2, col2, ef0, ef1, zrows, ones1k)


def _conv_scatter_body(nc, ch, row_hbm, col_hbm, *rest):
    u_hbms = rest[:nc]
    z_hbm = rest[nc]
    sp_hbms = rest[nc + 1:2 * nc + 1]
    acc, irow, icol, upd, sem = rest[2 * nc + 1:]
    s = lax.axis_index("s")
    w = s * 2 + ch
    base_r = w * PR

    for chunk in range(nc):
        pltpu.sync_copy(z_hbm, acc.at[pl.ds(s * TZ, TZ), :])
        plsc.subcore_barrier()

        def batch(g, carry, u_hbm=u_hbms[chunk]):
            r0 = base_r + g * 8
            pltpu.sync_copy(row_hbm.at[pl.ds(r0, 8), :], irow)
            pltpu.sync_copy(col_hbm.at[pl.ds(r0, 8), :], icol)
            cps = [pltpu.async_copy(u_hbm.at[irow.at[j]],
                                    upd.at[pl.ds(j * 128, 128), :], sem)
                   for j in range(8)]
            for cp in cps:
                cp.wait()
            for j in range(8):
                pltpu.sync_copy(upd.at[pl.ds(j * 128, 128), :],
                                acc.at[icol.at[j]], add=True)
            return carry

        lax.fori_loop(0, NB, batch, 0)
        plsc.subcore_barrier()
        pltpu.sync_copy(acc.at[pl.ds(s * TW, TW), :],
                        sp_hbms[chunk].at[pl.ds(s * TW, TW), :])
        plsc.subcore_barrier()


def _conv_scatter(ch, row2, col2, u_list, zrows):
    nc = len(u_list)
    mesh = plsc.VectorSubcoreMesh(core_axis_name="c", subcore_axis_name="s",
                                  num_cores=1)
    return pl.kernel(
        functools.partial(_conv_scatter_body, nc, ch),
        out_type=[jax.ShapeDtypeStruct((NP, 16), _f32) for _ in range(nc)],
        mesh=mesh,
        compiler_params=pltpu.CompilerParams(use_tc_tiling_on_sc=False),
        scratch_types=[
            pltpu.VMEM_SHARED((NA, 16), _f32),
            pltpu.VMEM((8, 128), jnp.int32),
            pltpu.VMEM((8, 128), jnp.int32),
            pltpu.VMEM((1024, 16), _f32),
            pltpu.SemaphoreType.DMA,
        ],
    )(row2, col2, *u_list, zrows)


def _node1_body(x_ref, sp0_ref, sp1_ref, degp_ref, wg1_ref,
                dinv_ref, *u_refs):
    s0 = sp0_ref[0] + sp0_ref[1]
    s1 = sp1_ref[0] + sp1_ref[1]
    cnt = jnp.maximum(s1[:, 14:15], 1.0)
    sums30 = jnp.concatenate([s0, s1[:, :14]], axis=1)
    h0 = x_ref[...] + sums30 / cnt
    deg = 1.0 + degp_ref[0] + degp_ref[1]
    dinv = lax.rsqrt(deg)
    dinv_ref[...] = dinv
    hw = jnp.dot(h0, wg1_ref[...], preferred_element_type=_f32)
    u = hw * dinv[:, 0:1]
    for cidx in range(8):
        u_refs[cidx][...] = u[:, cidx * 16:(cidx + 1) * 16]


def _node1(x, sp0, sp1, degp, W_g1):
    return pl.pallas_call(
        _node1_body,
        grid=(GN,),
        in_specs=[
            pl.BlockSpec((BN, 30), lambda i: (i, 0)),
            pl.BlockSpec((2, BN, 16), lambda i: (0, i, 0)),
            pl.BlockSpec((2, BN, 16), lambda i: (0, i, 0)),
            pl.BlockSpec((2, BN, 16), lambda i: (0, i, 0)),
            pl.BlockSpec((30, 128), lambda i: (0, 0)),
        ],
        out_specs=[pl.BlockSpec((BN, 16), lambda i: (i, 0))] * 9,
        out_shape=[jax.ShapeDtypeStruct((NP, 16), _f32)] * 9,
    )(x, sp0, sp1, degp, W_g1)


def _node2_body(dinv_ref, bg1_ref, wg2_ref, *refs):
    sp_refs = refs[:8]
    u1_refs = refs[8:16]
    u2_refs = refs[16:]
    d1 = dinv_ref[...][:, 0:1]
    cols = []
    for cidx in range(8):
        t = (sp_refs[cidx][0] + sp_refs[cidx][1] + u1_refs[cidx][...]) * d1
        cols.append(jnp.maximum(t + bg1_ref[:, cidx * 16:(cidx + 1) * 16], 0.0))
    h1 = jnp.concatenate(cols, axis=1)
    hw = jnp.dot(h1, wg2_ref[...], preferred_element_type=_f32)
    u2 = hw * d1
    for cidx in range(4):
        u2_refs[cidx][...] = u2[:, cidx * 16:(cidx + 1) * 16]


def _node2(dinv, b_g1, W_g2, sp_list, u1_list):
    return pl.pallas_call(
        _node2_body,
        grid=(GN,),
        in_specs=[
            pl.BlockSpec((BN, 16), lambda i: (i, 0)),
            pl.BlockSpec((1, 128), lambda i: (0, 0)),
            pl.BlockSpec((128, 64), lambda i: (0, 0)),
        ] + [pl.BlockSpec((2, BN, 16), lambda i: (0, i, 0))] * 8
          + [pl.BlockSpec((BN, 16), lambda i: (i, 0))] * 8,
        out_specs=[pl.BlockSpec((BN, 16), lambda i: (i, 0))] * 4,
        out_shape=[jax.ShapeDtypeStruct((NP, 16), _f32)] * 4,
    )(dinv, b_g1, W_g2, *sp_list, *u1_list)


def _head_body(dinv_ref, bg2_ref, wa_ref, ba_ref, wo_ref, bo_ref, *refs):
    sp_refs = refs[:4]
    u2_refs = refs[4:8]
    psum_ref, out_ref = refs[8:]
    i = pl.program_id(0)
    d1 = dinv_ref[...][:, 0:1]
    cols = []
    for cidx in range(4):
        t = (sp_refs[cidx][0] + sp_refs[cidx][1] + u2_refs[cidx][...]) * d1
        cols.append(jnp.maximum(t + bg2_ref[:, cidx * 16:(cidx + 1) * 16], 0.0))
    h2 = jnp.concatenate(cols, axis=1)
    a = jnp.maximum(
        jnp.dot(h2, wa_ref[...], preferred_element_type=_f32) + ba_ref[...],
        0.0)
    rid = i * BN + lax.broadcasted_iota(jnp.int32, (BN, 1), 0)
    a = jnp.where(rid < N, a, 0.0)
    part = jnp.sum(a, axis=0, keepdims=True)

    @pl.when(i == 0)
    def _():
        psum_ref[...] = part

    @pl.when(i > 0)
    def _():
        psum_ref[...] = psum_ref[...] + part

    @pl.when(i == GN - 1)
    def _():
        out_ref[...] = jnp.dot(psum_ref[...] * (1.0 / N), wo_ref[...],
                               preferred_element_type=_f32) + bo_ref[...]


def _head(dinv, b_g2, W_a, b_a, W_o, b_o, sp_list, u2_list):
    return pl.pallas_call(
        _head_body,
        grid=(GN,),
        in_specs=[
            pl.BlockSpec((BN, 16), lambda i: (i, 0)),
            pl.BlockSpec((1, 64), lambda i: (0, 0)),
            pl.BlockSpec((64, 64), lambda i: (0, 0)),
            pl.BlockSpec((1, 64), lambda i: (0, 0)),
            pl.BlockSpec((64, 16), lambda i: (0, 0)),
            pl.BlockSpec((1, 16), lambda i: (0, 0)),
        ] + [pl.BlockSpec((2, BN, 16), lambda i: (0, i, 0))] * 4
          + [pl.BlockSpec((BN, 16), lambda i: (i, 0))] * 4,
        out_specs=[pl.BlockSpec((1, 64), lambda i: (0, 0)),
                   pl.BlockSpec((1, 16), lambda i: (0, 0))],
        out_shape=[jax.ShapeDtypeStruct((1, 64), _f32),
                   jax.ShapeDtypeStruct((1, 16), _f32)],
    )(dinv, b_g2, W_a, b_a, W_o, b_o, *sp_list, *u2_list)


def kernel(x, edge_index, edge_attr, W_e1, b_e1, W_e2, b_e2,
           W_g1, b_g1, W_g2, b_g2, W_a, b_a, W_o, b_o):
    row = edge_index[0]
    col = edge_index[1]
    npad = EP - E
    junk = jnp.full((npad,), NP, jnp.int32)
    row_g = jnp.concatenate([row, jnp.zeros((npad,), jnp.int32)]).reshape(ER, 128)
    src_s = jnp.concatenate([row, junk]).reshape(ER, 128)
    col_s = jnp.concatenate([col, junk]).reshape(ER, 128)
    zrows = jnp.zeros((TZ, 16), _f32)
    ones1k = jnp.ones((1024, 16), _f32)

    W_e2p = jnp.zeros((128, 32), _f32).at[:, :30].set(W_e2)
    b_e2p = jnp.zeros((32,), _f32).at[:30].set(b_e2).at[30].set(1.0)
    ef0, ef1 = _edge_mlp(edge_attr, W_e1, b_e1[None, :], W_e2p, b_e2p[None, :])

    agg0 = _edge_agg(0, src_s, col_s, ef0, ef1, zrows, ones1k)
    agg1 = _edge_agg(1, src_s, col_s, ef0, ef1, zrows, ones1k)
    sp0 = jnp.stack([agg0[0], agg1[0]])
    sp1 = jnp.stack([agg0[1], agg1[1]])
    degp = jnp.stack([agg0[2], agg1[2]])

    n1 = _node1(x, sp0, sp1, degp, W_g1)
    dinv, u1_list = n1[0], list(n1[1:])

    s1a = _conv_scatter(0, row_g, col_s, u1_list, zrows)
    s1b = _conv_scatter(1, row_g, col_s, u1_list, zrows)
    s1p = [jnp.stack([a, b]) for a, b in zip(s1a, s1b)]

    u2_list = list(_node2(dinv, b_g1[None, :], W_g2, s1p, u1_list))

    s2a = _conv_scatter(0, row_g, col_s, u2_list, zrows)
    s2b = _conv_scatter(1, row_g, col_s, u2_list, zrows)
    s2p = [jnp.stack([a, b]) for a, b in zip(s2a, s2b)]

    _, out = _head(dinv, b_g2[None, :], W_a, b_a[None, :], W_o, b_o[None, :],
                   s2p, u2_list)
    return out[0]


# speedup vs baseline: 1.4676x; 1.4676x over previous
import functools

import jax
import jax.numpy as jnp
from jax import lax
from jax.experimental import pallas as pl
from jax.experimental.pallas import tpu as pltpu
from jax.experimental.pallas import tpu_sc as plsc

N = 100000
E = 1600000

NP = 100352
NA = NP + 128
NT = 16
NW = 32
TZ = NA // NT
TW = NP // NT

EP = 1605632
ER = EP // 128
PT = EP // NW
PR = PT // 128
NB = PR // 4

BE = 8192
BN = 1024
GN = NP // BN

_f32 = jnp.float32


def _edge_mlp_body(ea_ref, w1_ref, b1_ref, w2_ref, b2_ref, ef0_ref, ef1_ref):
    t = jnp.maximum(
        jnp.dot(ea_ref[...], w1_ref[...], preferred_element_type=_f32)
        + b1_ref[...], 0.0)
    ef = jnp.maximum(
        jnp.dot(t, w2_ref[...], preferred_element_type=_f32)
        + b2_ref[...], 0.0)
    ef0_ref[...] = ef[:, :16]
    ef1_ref[...] = ef[:, 16:]


def _edge_mlp(edge_attr, W_e1, b_e1, W_e2p, b_e2p):
    return pl.pallas_call(
        _edge_mlp_body,
        grid=(EP // BE,),
        in_specs=[
            pl.BlockSpec((BE, 16), lambda i: (i, 0)),
            pl.BlockSpec((16, 128), lambda i: (0, 0)),
            pl.BlockSpec((1, 128), lambda i: (0, 0)),
            pl.BlockSpec((128, 32), lambda i: (0, 0)),
            pl.BlockSpec((1, 32), lambda i: (0, 0)),
        ],
        out_specs=[
            pl.BlockSpec((BE, 16), lambda i: (i, 0)),
            pl.BlockSpec((BE, 16), lambda i: (i, 0)),
        ],
        out_shape=[
            jax.ShapeDtypeStruct((EP, 16), _f32),
            jax.ShapeDtypeStruct((EP, 16), _f32),
        ],
    )(edge_attr, W_e1, b_e1, W_e2p, b_e2p)


def _edge_agg_body(src_hbm, col_hbm, ef0_hbm, ef1_hbm, z_hbm, ones_hbm,
                   sp0_hbm, sp1_hbm, degp_hbm,
                   acc, isrc, upd2, ssem0, ssem1):
    c = lax.axis_index("c")
    s = lax.axis_index("s")
    w = s * 2 + c
    base_r = w * PR
    ssems = (ssem0, ssem1)
    zsrc = z_hbm.at[pl.ds(0, 512), :]

    for idx_hbm, ef_hbm, sp_hbm in ((src_hbm, ef0_hbm, sp0_hbm),
                                    (src_hbm, ef1_hbm, sp1_hbm),
                                    (col_hbm, None, degp_hbm)):
        pltpu.sync_copy(z_hbm, acc.at[pl.ds(s * TZ, TZ), :])
        if ef_hbm is None:
            pltpu.sync_copy(ones_hbm, upd2.at[0, :, :])
        plsc.subcore_barrier()

        def do_batch(bi, p, drain, idx_hbm=idx_hbm, ef_hbm=ef_hbm):
            r0 = base_r + bi * 4
            if drain:
                pltpu.make_async_copy(zsrc, upd2.at[p, :, :], ssems[p]).wait()
            pltpu.sync_copy(idx_hbm.at[pl.ds(r0, 4), :], isrc)
            if ef_hbm is not None:
                pltpu.sync_copy(ef_hbm.at[pl.ds(r0 * 128, 512), :],
                                upd2.at[p, :, :])
            for j in range(4):
                pltpu.async_copy(upd2.at[p, pl.ds(j * 128, 128), :],
                                 acc.at[isrc.at[j]], ssems[p], add=True)

        if ef_hbm is not None:
            do_batch(0, 0, False)
            do_batch(1, 1, False)

            def dbl(g, carry):
                do_batch(2 + 2 * g, 0, True)
                do_batch(3 + 2 * g, 1, True)
                return carry

            lax.fori_loop(0, (NB - 2) // 2, dbl, 0)
            pltpu.make_async_copy(zsrc, upd2.at[1, :, :], ssem1).wait()
            pltpu.make_async_copy(zsrc, upd2.at[0, :, :], ssem0).wait()
        else:
            def onesb(g, carry):
                do_batch(g, 0, False)
                return carry

            lax.fori_loop(0, NB, onesb, 0)

            def draino(g, carry):
                pltpu.make_async_copy(zsrc, upd2.at[0, :, :], ssem0).wait()
                return carry

            lax.fori_loop(0, NB, draino, 0)
        plsc.subcore_barrier()
        pltpu.sync_copy(acc.at[pl.ds(s * TW, TW), :],
                        sp_hbm.at[pl.ds(c * NP + s * TW, TW), :])
        plsc.subcore_barrier()


def _edge_agg(src2, col2, ef0, ef1, zrows, ones1k):
    mesh = plsc.VectorSubcoreMesh(core_axis_name="c", subcore_axis_name="s")
    return pl.kernel(
        _edge_agg_body,
        out_type=[
            jax.ShapeDtypeStruct((2 * NP, 16), _f32),
            jax.ShapeDtypeStruct((2 * NP, 16), _f32),
            jax.ShapeDtypeStruct((2 * NP, 16), _f32),
        ],
        mesh=mesh,
        compiler_params=pltpu.CompilerParams(use_tc_tiling_on_sc=False),
        scratch_types=[
            pltpu.VMEM_SHARED((NA, 16), _f32),
            pltpu.VMEM((4, 128), jnp.int32),
            pltpu.VMEM((2, 512, 16), _f32),
            pltpu.SemaphoreType.DMA,
            pltpu.SemaphoreType.DMA,
        ],
    )(src2, col2, ef0, ef1, zrows, ones1k)


def _conv_scatter_body(nc, row_hbm, col_hbm, *rest):
    u_hbms = rest[:nc]
    z_hbm = rest[nc]
    sp_hbms = rest[nc + 1:2 * nc + 1]
    acc, irow, icol2, upd2, gsem, ssem0, ssem1 = rest[2 * nc + 1:]
    c = lax.axis_index("c")
    s = lax.axis_index("s")
    w = s * 2 + c
    base_r = w * PR
    ssems = (ssem0, ssem1)
    zsrc = z_hbm.at[pl.ds(0, 512), :]

    for chunk in range(nc):
        pltpu.sync_copy(z_hbm, acc.at[pl.ds(s * TZ, TZ), :])
        plsc.subcore_barrier()

        def do_batch(bi, p, drain, u_hbm=u_hbms[chunk]):
            r0 = base_r + bi * 4
            if drain:
                pltpu.make_async_copy(zsrc, upd2.at[p, :, :], ssems[p]).wait()
            pltpu.sync_copy(row_hbm.at[pl.ds(r0, 4), :], irow)
            pltpu.sync_copy(col_hbm.at[pl.ds(r0, 4), :], icol2.at[p, :, :])
            cps = [pltpu.async_copy(u_hbm.at[irow.at[j]],
                                    upd2.at[p, pl.ds(j * 128, 128), :], gsem)
                   for j in range(4)]
            for cp in cps:
                cp.wait()
            for j in range(4):
                pltpu.async_copy(upd2.at[p, pl.ds(j * 128, 128), :],
                                 acc.at[icol2.at[p, j]], ssems[p], add=True)

        do_batch(0, 0, False)
        do_batch(1, 1, False)

        def dbl(g, carry):
            do_batch(2 + 2 * g, 0, True)
            do_batch(3 + 2 * g, 1, True)
            return carry

        lax.fori_loop(0, (NB - 2) // 2, dbl, 0)
        pltpu.make_async_copy(zsrc, upd2.at[1, :, :], ssem1).wait()
        pltpu.make_async_copy(zsrc, upd2.at[0, :, :], ssem0).wait()
        plsc.subcore_barrier()
        pltpu.sync_copy(acc.at[pl.ds(s * TW, TW), :],
                        sp_hbms[chunk].at[pl.ds(c * NP + s * TW, TW), :])
        plsc.subcore_barrier()


def _conv_scatter(row2, col2, u_list, zrows):
    nc = len(u_list)
    mesh = plsc.VectorSubcoreMesh(core_axis_name="c", subcore_axis_name="s")
    return pl.kernel(
        functools.partial(_conv_scatter_body, nc),
        out_type=[jax.ShapeDtypeStruct((2 * NP, 16), _f32) for _ in range(nc)],
        mesh=mesh,
        compiler_params=pltpu.CompilerParams(use_tc_tiling_on_sc=False),
        scratch_types=[
            pltpu.VMEM_SHARED((NA, 16), _f32),
            pltpu.VMEM((4, 128), jnp.int32),
            pltpu.VMEM((2, 4, 128), jnp.int32),
            pltpu.VMEM((2, 512, 16), _f32),
            pltpu.SemaphoreType.DMA,
            pltpu.SemaphoreType.DMA,
            pltpu.SemaphoreType.DMA,
        ],
    )(row2, col2, *u_list, zrows)


def _node1_body(x_ref, sp0_ref, sp1_ref, degp_ref, wg1_ref,
                dinv_ref, *u_refs):
    s0 = sp0_ref[0] + sp0_ref[1]
    s1 = sp1_ref[0] + sp1_ref[1]
    cnt = jnp.maximum(s1[:, 14:15], 1.0)
    sums30 = jnp.concatenate([s0, s1[:, :14]], axis=1)
    h0 = x_ref[...] + sums30 / cnt
    deg = 1.0 + degp_ref[0] + degp_ref[1]
    dinv = lax.rsqrt(deg)
    dinv_ref[...] = dinv
    hw = jnp.dot(h0, wg1_ref[...], preferred_element_type=_f32)
    u = hw * dinv[:, 0:1]
    for cidx in range(8):
        u_refs[cidx][...] = u[:, cidx * 16:(cidx + 1) * 16]


def _node1(x, sp0, sp1, degp, W_g1):
    return pl.pallas_call(
        _node1_body,
        grid=(GN,),
        in_specs=[
            pl.BlockSpec((BN, 30), lambda i: (i, 0)),
            pl.BlockSpec((2, BN, 16), lambda i: (0, i, 0)),
            pl.BlockSpec((2, BN, 16), lambda i: (0, i, 0)),
            pl.BlockSpec((2, BN, 16), lambda i: (0, i, 0)),
            pl.BlockSpec((30, 128), lambda i: (0, 0)),
        ],
        out_specs=[pl.BlockSpec((BN, 16), lambda i: (i, 0))] * 9,
        out_shape=[jax.ShapeDtypeStruct((NP, 16), _f32)] * 9,
    )(x, sp0, sp1, degp, W_g1)


def _node2_body(dinv_ref, bg1_ref, wg2_ref, *refs):
    sp_refs = refs[:8]
    u1_refs = refs[8:16]
    u2_refs = refs[16:]
    d1 = dinv_ref[...][:, 0:1]
    cols = []
    for cidx in range(8):
        t = (sp_refs[cidx][0] + sp_refs[cidx][1] + u1_refs[cidx][...]) * d1
        cols.append(jnp.maximum(t + bg1_ref[:, cidx * 16:(cidx + 1) * 16], 0.0))
    h1 = jnp.concatenate(cols, axis=1)
    hw = jnp.dot(h1, wg2_ref[...], preferred_element_type=_f32)
    u2 = hw * d1
    for cidx in range(4):
        u2_refs[cidx][...] = u2[:, cidx * 16:(cidx + 1) * 16]


def _node2(dinv, b_g1, W_g2, sp_list, u1_list):
    return pl.pallas_call(
        _node2_body,
        grid=(GN,),
        in_specs=[
            pl.BlockSpec((BN, 16), lambda i: (i, 0)),
            pl.BlockSpec((1, 128), lambda i: (0, 0)),
            pl.BlockSpec((128, 64), lambda i: (0, 0)),
        ] + [pl.BlockSpec((2, BN, 16), lambda i: (0, i, 0))] * 8
          + [pl.BlockSpec((BN, 16), lambda i: (i, 0))] * 8,
        out_specs=[pl.BlockSpec((BN, 16), lambda i: (i, 0))] * 4,
        out_shape=[jax.ShapeDtypeStruct((NP, 16), _f32)] * 4,
    )(dinv, b_g1, W_g2, *sp_list, *u1_list)


def _head_body(dinv_ref, bg2_ref, wa_ref, ba_ref, wo_ref, bo_ref, *refs):
    sp_refs = refs[:4]
    u2_refs = refs[4:8]
    psum_ref, out_ref = refs[8:]
    i = pl.program_id(0)
    d1 = dinv_ref[...][:, 0:1]
    cols = []
    for cidx in range(4):
        t = (sp_refs[cidx][0] + sp_refs[cidx][1] + u2_refs[cidx][...]) * d1
        cols.append(jnp.maximum(t + bg2_ref[:, cidx * 16:(cidx + 1) * 16], 0.0))
    h2 = jnp.concatenate(cols, axis=1)
    a = jnp.maximum(
        jnp.dot(h2, wa_ref[...], preferred_element_type=_f32) + ba_ref[...],
        0.0)
    rid = i * BN + lax.broadcasted_iota(jnp.int32, (BN, 1), 0)
    a = jnp.where(rid < N, a, 0.0)
    part = jnp.sum(a, axis=0, keepdims=True)

    @pl.when(i == 0)
    def _():
        psum_ref[...] = part

    @pl.when(i > 0)
    def _():
        psum_ref[...] = psum_ref[...] + part

    @pl.when(i == GN - 1)
    def _():
        out_ref[...] = jnp.dot(psum_ref[...] * (1.0 / N), wo_ref[...],
                               preferred_element_type=_f32) + bo_ref[...]


def _head(dinv, b_g2, W_a, b_a, W_o, b_o, sp_list, u2_list):
    return pl.pallas_call(
        _head_body,
        grid=(GN,),
        in_specs=[
            pl.BlockSpec((BN, 16), lambda i: (i, 0)),
            pl.BlockSpec((1, 64), lambda i: (0, 0)),
            pl.BlockSpec((64, 64), lambda i: (0, 0)),
            pl.BlockSpec((1, 64), lambda i: (0, 0)),
            pl.BlockSpec((64, 16), lambda i: (0, 0)),
            pl.BlockSpec((1, 16), lambda i: (0, 0)),
        ] + [pl.BlockSpec((2, BN, 16), lambda i: (0, i, 0))] * 4
          + [pl.BlockSpec((BN, 16), lambda i: (i, 0))] * 4,
        out_specs=[pl.BlockSpec((1, 64), lambda i: (0, 0)),
                   pl.BlockSpec((1, 16), lambda i: (0, 0))],
        out_shape=[jax.ShapeDtypeStruct((1, 64), _f32),
                   jax.ShapeDtypeStruct((1, 16), _f32)],
    )(dinv, b_g2, W_a, b_a, W_o, b_o, *sp_list, *u2_list)


def kernel(x, edge_index, edge_attr, W_e1, b_e1, W_e2, b_e2,
           W_g1, b_g1, W_g2, b_g2, W_a, b_a, W_o, b_o):
    row = edge_index[0]
    col = edge_index[1]
    npad = EP - E
    junk = jnp.full((npad,), NP, jnp.int32)
    row_g = jnp.concatenate([row, jnp.zeros((npad,), jnp.int32)]).reshape(ER, 128)
    src_s = jnp.concatenate([row, junk]).reshape(ER, 128)
    col_s = jnp.concatenate([col, junk]).reshape(ER, 128)
    zrows = jnp.zeros((TZ, 16), _f32)
    ones1k = jnp.ones((512, 16), _f32)

    W_e2p = jnp.zeros((128, 32), _f32).at[:, :30].set(W_e2)
    b_e2p = jnp.zeros((32,), _f32).at[:30].set(b_e2).at[30].set(1.0)
    ef0, ef1 = _edge_mlp(edge_attr, W_e1, b_e1[None, :], W_e2p, b_e2p[None, :])

    sp0, sp1, degp = _edge_agg(src_s, col_s, ef0, ef1, zrows, ones1k)
    sp0 = sp0.reshape(2, NP, 16)
    sp1 = sp1.reshape(2, NP, 16)
    degp = degp.reshape(2, NP, 16)

    n1 = _node1(x, sp0, sp1, degp, W_g1)
    dinv, u1_list = n1[0], list(n1[1:])

    s1p = _conv_scatter(row_g, col_s, u1_list, zrows)
    s1p = [s.reshape(2, NP, 16) for s in s1p]

    u2_list = list(_node2(dinv, b_g1[None, :], W_g2, s1p, u1_list))

    s2p = _conv_scatter(row_g, col_s, u2_list, zrows)
    s2p = [s.reshape(2, NP, 16) for s in s2p]

    _, out = _head(dinv, b_g2[None, :], W_a, b_a[None, :], W_o, b_o[None, :],
                   s2p, u2_list)
    return out[0]


# speedup vs baseline: 1.6822x; 1.1462x over previous
import functools

import jax
import jax.numpy as jnp
from jax import lax
from jax.experimental import pallas as pl
from jax.experimental.pallas import tpu as pltpu
from jax.experimental.pallas import tpu_sc as plsc

N = 100000
E = 1600000

NP = 100352
NA = NP + 128
NT = 16
NW = 32
TZ = NA // NT
TW = NP // NT

EP = 1605632
ER = EP // 128
PT = EP // NW
PR = PT // 128
NB = PR // 4

BE = 8192
BN = 1024
GN = NP // BN

_f32 = jnp.float32


def _edge_mlp_body(ea_ref, w1_ref, b1_ref, w2_ref, b2_ref, ef0_ref, ef1_ref):
    t = jnp.maximum(
        jnp.dot(ea_ref[...], w1_ref[...], preferred_element_type=_f32)
        + b1_ref[...], 0.0)
    ef = jnp.maximum(
        jnp.dot(t, w2_ref[...], preferred_element_type=_f32)
        + b2_ref[...], 0.0)
    ef0_ref[...] = ef[:, :16]
    ef1_ref[...] = ef[:, 16:]


def _edge_mlp(edge_attr, W_e1, b_e1, W_e2p, b_e2p):
    return pl.pallas_call(
        _edge_mlp_body,
        grid=(EP // BE,),
        in_specs=[
            pl.BlockSpec((BE, 16), lambda i: (i, 0)),
            pl.BlockSpec((16, 128), lambda i: (0, 0)),
            pl.BlockSpec((1, 128), lambda i: (0, 0)),
            pl.BlockSpec((128, 32), lambda i: (0, 0)),
            pl.BlockSpec((1, 32), lambda i: (0, 0)),
        ],
        out_specs=[
            pl.BlockSpec((BE, 16), lambda i: (i, 0)),
            pl.BlockSpec((BE, 16), lambda i: (i, 0)),
        ],
        out_shape=[
            jax.ShapeDtypeStruct((EP, 16), _f32),
            jax.ShapeDtypeStruct((EP, 16), _f32),
        ],
    )(edge_attr, W_e1, b_e1, W_e2p, b_e2p)


def _edge_agg_body(src_hbm, col_hbm, ef0_hbm, ef1_hbm, z_hbm, ones_hbm,
                   sp0_hbm, sp1_hbm, degp_hbm,
                   acc, isrc, upd2, ssem0, ssem1):
    c = lax.axis_index("c")
    s = lax.axis_index("s")
    w = s * 2 + c
    base_r = w * PR
    ssems = (ssem0, ssem1)
    zsrc = z_hbm.at[pl.ds(0, 512), :]

    for idx_hbm, ef_hbm, sp_hbm in ((src_hbm, ef0_hbm, sp0_hbm),
                                    (src_hbm, ef1_hbm, sp1_hbm),
                                    (col_hbm, None, degp_hbm)):
        pltpu.sync_copy(z_hbm, acc.at[pl.ds(s * TZ, TZ), :])
        if ef_hbm is None:
            pltpu.sync_copy(ones_hbm, upd2.at[0, :, :])
        plsc.subcore_barrier()

        def do_batch(bi, p, drain, idx_hbm=idx_hbm, ef_hbm=ef_hbm):
            r0 = base_r + bi * 4
            if drain:
                pltpu.make_async_copy(zsrc, upd2.at[p, :, :], ssems[p]).wait()
            pltpu.sync_copy(idx_hbm.at[pl.ds(r0, 4), :], isrc)
            if ef_hbm is not None:
                pltpu.sync_copy(ef_hbm.at[pl.ds(r0 * 128, 512), :],
                                upd2.at[p, :, :])
            for j in range(4):
                pltpu.async_copy(upd2.at[p, pl.ds(j * 128, 128), :],
                                 acc.at[isrc.at[j]], ssems[p], add=True)

        if ef_hbm is not None:
            do_batch(0, 0, False)
            do_batch(1, 1, False)

            def dbl(g, carry):
                do_batch(2 + 2 * g, 0, True)
                do_batch(3 + 2 * g, 1, True)
                return carry

            lax.fori_loop(0, (NB - 2) // 2, dbl, 0)
            pltpu.make_async_copy(zsrc, upd2.at[1, :, :], ssem1).wait()
            pltpu.make_async_copy(zsrc, upd2.at[0, :, :], ssem0).wait()
        else:
            def onesb(g, carry):
                do_batch(g, 0, False)
                return carry

            lax.fori_loop(0, NB, onesb, 0)

            def draino(g, carry):
                pltpu.make_async_copy(zsrc, upd2.at[0, :, :], ssem0).wait()
                return carry

            lax.fori_loop(0, NB, draino, 0)
        plsc.subcore_barrier()
        pltpu.sync_copy(acc.at[pl.ds(s * TW, TW), :],
                        sp_hbm.at[pl.ds(c * NP + s * TW, TW), :])
        plsc.subcore_barrier()


def _edge_agg(src2, col2, ef0, ef1, zrows, ones1k):
    mesh = plsc.VectorSubcoreMesh(core_axis_name="c", subcore_axis_name="s")
    return pl.kernel(
        _edge_agg_body,
        out_type=[
            jax.ShapeDtypeStruct((2 * NP, 16), _f32),
            jax.ShapeDtypeStruct((2 * NP, 16), _f32),
            jax.ShapeDtypeStruct((2 * NP, 16), _f32),
        ],
        mesh=mesh,
        compiler_params=pltpu.CompilerParams(use_tc_tiling_on_sc=False),
        scratch_types=[
            pltpu.VMEM_SHARED((NA, 16), _f32),
            pltpu.VMEM((4, 128), jnp.int32),
            pltpu.VMEM((2, 512, 16), _f32),
            pltpu.SemaphoreType.DMA,
            pltpu.SemaphoreType.DMA,
        ],
    )(src2, col2, ef0, ef1, zrows, ones1k)


def _conv_scatter_body(nc, rc_hbm, *rest):
    u_hbms = rest[:nc]
    z_hbm = rest[nc]
    sp_hbms = rest[nc + 1:2 * nc + 1]
    acc, idxb, upd2, gsem, isem, ssem0, ssem1 = rest[2 * nc + 1:]
    c = lax.axis_index("c")
    s = lax.axis_index("s")
    w = s * 2 + c
    base_rc = (w * PR // 4) * 8
    ssems = (ssem0, ssem1)
    zsrc = z_hbm.at[pl.ds(0, 512), :]
    zidx = rc_hbm.at[pl.ds(0, 8), :]

    for chunk in range(nc):
        pltpu.sync_copy(z_hbm, acc.at[pl.ds(s * TZ, TZ), :])
        plsc.subcore_barrier()

        def do_batch(bi, p, pre, dso, first=False, u_hbm=u_hbms[chunk]):
            if first:
                pltpu.sync_copy(rc_hbm.at[pl.ds(base_rc, 8), :],
                                idxb.at[p, :, :])
            else:
                pltpu.make_async_copy(zidx, idxb.at[p, :, :], isem).wait()
            cps = [pltpu.async_copy(u_hbm.at[idxb.at[p, j]],
                                    upd2.at[p, pl.ds(j * 128, 128), :], gsem)
                   for j in range(4)]
            if dso:
                pltpu.make_async_copy(zsrc, upd2.at[1 - p, :, :],
                                      ssems[1 - p]).wait()
            if pre:
                pltpu.async_copy(rc_hbm.at[pl.ds(base_rc + (bi + 1) * 8, 8), :],
                                 idxb.at[1 - p, :, :], isem)
            for cp in cps:
                cp.wait()
            for j in range(4):
                pltpu.async_copy(upd2.at[p, pl.ds(j * 128, 128), :],
                                 acc.at[idxb.at[p, 4 + j]], ssems[p], add=True)

        do_batch(0, 0, True, False, first=True)

        def dbl(g, carry):
            do_batch(1 + 2 * g, 1, True, True)
            do_batch(2 + 2 * g, 0, True, True)
            return carry

        lax.fori_loop(0, (NB - 2) // 2, dbl, 0)
        do_batch(NB - 1, 1, False, True)
        pltpu.make_async_copy(zsrc, upd2.at[1, :, :], ssem1).wait()
        plsc.subcore_barrier()
        pltpu.sync_copy(acc.at[pl.ds(s * TW, TW), :],
                        sp_hbms[chunk].at[pl.ds(c * NP + s * TW, TW), :])
        plsc.subcore_barrier()


def _conv_scatter(rc, u_list, zrows):
    nc = len(u_list)
    mesh = plsc.VectorSubcoreMesh(core_axis_name="c", subcore_axis_name="s")
    return pl.kernel(
        functools.partial(_conv_scatter_body, nc),
        out_type=[jax.ShapeDtypeStruct((2 * NP, 16), _f32) for _ in range(nc)],
        mesh=mesh,
        compiler_params=pltpu.CompilerParams(use_tc_tiling_on_sc=False),
        scratch_types=[
            pltpu.VMEM_SHARED((NA, 16), _f32),
            pltpu.VMEM((2, 8, 128), jnp.int32),
            pltpu.VMEM((2, 512, 16), _f32),
            pltpu.SemaphoreType.DMA,
            pltpu.SemaphoreType.DMA,
            pltpu.SemaphoreType.DMA,
            pltpu.SemaphoreType.DMA,
        ],
    )(rc, *u_list, zrows)


def _node1_body(x_ref, sp0_ref, sp1_ref, degp_ref, wg1_ref,
                dinv_ref, *u_refs):
    s0 = sp0_ref[0] + sp0_ref[1]
    s1 = sp1_ref[0] + sp1_ref[1]
    cnt = jnp.maximum(s1[:, 14:15], 1.0)
    sums30 = jnp.concatenate([s0, s1[:, :14]], axis=1)
    h0 = x_ref[...] + sums30 / cnt
    deg = 1.0 + degp_ref[0] + degp_ref[1]
    dinv = lax.rsqrt(deg)
    dinv_ref[...] = dinv
    hw = jnp.dot(h0, wg1_ref[...], preferred_element_type=_f32)
    u = hw * dinv[:, 0:1]
    for cidx in range(8):
        u_refs[cidx][...] = u[:, cidx * 16:(cidx + 1) * 16]


def _node1(x, sp0, sp1, degp, W_g1):
    return pl.pallas_call(
        _node1_body,
        grid=(GN,),
        in_specs=[
            pl.BlockSpec((BN, 30), lambda i: (i, 0)),
            pl.BlockSpec((2, BN, 16), lambda i: (0, i, 0)),
            pl.BlockSpec((2, BN, 16), lambda i: (0, i, 0)),
            pl.BlockSpec((2, BN, 16), lambda i: (0, i, 0)),
            pl.BlockSpec((30, 128), lambda i: (0, 0)),
        ],
        out_specs=[pl.BlockSpec((BN, 16), lambda i: (i, 0))] * 9,
        out_shape=[jax.ShapeDtypeStruct((NP, 16), _f32)] * 9,
    )(x, sp0, sp1, degp, W_g1)


def _node2_body(dinv_ref, bg1_ref, wg2_ref, *refs):
    sp_refs = refs[:8]
    u1_refs = refs[8:16]
    u2_refs = refs[16:]
    d1 = dinv_ref[...][:, 0:1]
    cols = []
    for cidx in range(8):
        t = (sp_refs[cidx][0] + sp_refs[cidx][1] + u1_refs[cidx][...]) * d1
        cols.append(jnp.maximum(t + bg1_ref[:, cidx * 16:(cidx + 1) * 16], 0.0))
    h1 = jnp.concatenate(cols, axis=1)
    hw = jnp.dot(h1, wg2_ref[...], preferred_element_type=_f32)
    u2 = hw * d1
    for cidx in range(4):
        u2_refs[cidx][...] = u2[:, cidx * 16:(cidx + 1) * 16]


def _node2(dinv, b_g1, W_g2, sp_list, u1_list):
    return pl.pallas_call(
        _node2_body,
        grid=(GN,),
        in_specs=[
            pl.BlockSpec((BN, 16), lambda i: (i, 0)),
            pl.BlockSpec((1, 128), lambda i: (0, 0)),
            pl.BlockSpec((128, 64), lambda i: (0, 0)),
        ] + [pl.BlockSpec((2, BN, 16), lambda i: (0, i, 0))] * 8
          + [pl.BlockSpec((BN, 16), lambda i: (i, 0))] * 8,
        out_specs=[pl.BlockSpec((BN, 16), lambda i: (i, 0))] * 4,
        out_shape=[jax.ShapeDtypeStruct((NP, 16), _f32)] * 4,
    )(dinv, b_g1, W_g2, *sp_list, *u1_list)


def _head_body(dinv_ref, bg2_ref, wa_ref, ba_ref, wo_ref, bo_ref, *refs):
    sp_refs = refs[:4]
    u2_refs = refs[4:8]
    psum_ref, out_ref = refs[8:]
    i = pl.program_id(0)
    d1 = dinv_ref[...][:, 0:1]
    cols = []
    for cidx in range(4):
        t = (sp_refs[cidx][0] + sp_refs[cidx][1] + u2_refs[cidx][...]) * d1
        cols.append(jnp.maximum(t + bg2_ref[:, cidx * 16:(cidx + 1) * 16], 0.0))
    h2 = jnp.concatenate(cols, axis=1)
    a = jnp.maximum(
        jnp.dot(h2, wa_ref[...], preferred_element_type=_f32) + ba_ref[...],
        0.0)
    rid = i * BN + lax.broadcasted_iota(jnp.int32, (BN, 1), 0)
    a = jnp.where(rid < N, a, 0.0)
    part = jnp.sum(a, axis=0, keepdims=True)

    @pl.when(i == 0)
    def _():
        psum_ref[...] = part

    @pl.when(i > 0)
    def _():
        psum_ref[...] = psum_ref[...] + part

    @pl.when(i == GN - 1)
    def _():
        out_ref[...] = jnp.dot(psum_ref[...] * (1.0 / N), wo_ref[...],
                               preferred_element_type=_f32) + bo_ref[...]


def _head(dinv, b_g2, W_a, b_a, W_o, b_o, sp_list, u2_list):
    return pl.pallas_call(
        _head_body,
        grid=(GN,),
        in_specs=[
            pl.BlockSpec((BN, 16), lambda i: (i, 0)),
            pl.BlockSpec((1, 64), lambda i: (0, 0)),
            pl.BlockSpec((64, 64), lambda i: (0, 0)),
            pl.BlockSpec((1, 64), lambda i: (0, 0)),
            pl.BlockSpec((64, 16), lambda i: (0, 0)),
            pl.BlockSpec((1, 16), lambda i: (0, 0)),
        ] + [pl.BlockSpec((2, BN, 16), lambda i: (0, i, 0))] * 4
          + [pl.BlockSpec((BN, 16), lambda i: (i, 0))] * 4,
        out_specs=[pl.BlockSpec((1, 64), lambda i: (0, 0)),
                   pl.BlockSpec((1, 16), lambda i: (0, 0))],
        out_shape=[jax.ShapeDtypeStruct((1, 64), _f32),
                   jax.ShapeDtypeStruct((1, 16), _f32)],
    )(dinv, b_g2, W_a, b_a, W_o, b_o, *sp_list, *u2_list)


def kernel(x, edge_index, edge_attr, W_e1, b_e1, W_e2, b_e2,
           W_g1, b_g1, W_g2, b_g2, W_a, b_a, W_o, b_o):
    row = edge_index[0]
    col = edge_index[1]
    npad = EP - E
    junk = jnp.full((npad,), NP, jnp.int32)
    row_g = jnp.concatenate([row, jnp.zeros((npad,), jnp.int32)]).reshape(ER, 128)
    src_s = jnp.concatenate([row, junk]).reshape(ER, 128)
    col_s = jnp.concatenate([col, junk]).reshape(ER, 128)
    rc_il = jnp.concatenate([row_g.reshape(ER // 4, 1, 4, 128),
                             col_s.reshape(ER // 4, 1, 4, 128)],
                            axis=1).reshape(2 * ER, 128)
    zrows = jnp.zeros((TZ, 16), _f32)
    ones1k = jnp.ones((512, 16), _f32)

    W_e2p = jnp.zeros((128, 32), _f32).at[:, :30].set(W_e2)
    b_e2p = jnp.zeros((32,), _f32).at[:30].set(b_e2).at[30].set(1.0)
    ef0, ef1 = _edge_mlp(edge_attr, W_e1, b_e1[None, :], W_e2p, b_e2p[None, :])

    sp0, sp1, degp = _edge_agg(src_s, col_s, ef0, ef1, zrows, ones1k)
    sp0 = sp0.reshape(2, NP, 16)
    sp1 = sp1.reshape(2, NP, 16)
    degp = degp.reshape(2, NP, 16)

    n1 = _node1(x, sp0, sp1, degp, W_g1)
    dinv, u1_list = n1[0], list(n1[1:])

    s1p = _conv_scatter(rc_il, u1_list, zrows)
    s1p = [s.reshape(2, NP, 16) for s in s1p]

    u2_list = list(_node2(dinv, b_g1[None, :], W_g2, s1p, u1_list))

    s2p = _conv_scatter(rc_il, u2_list, zrows)
    s2p = [s.reshape(2, NP, 16) for s in s2p]

    _, out = _head(dinv, b_g2[None, :], W_a, b_a[None, :], W_o, b_o[None, :],
                   s2p, u2_list)
    return out[0]
